# baseline (speedup 1.0000x reference)
import functools

import jax
import jax.numpy as jnp
from jax import lax
from jax.experimental import pallas as pl
from jax.experimental.pallas import tpu as pltpu

GS = 256
TRUNC_VOX = 39
NARROW_VOX = 13
VS = 2.0 / GS
TRUNC = VS * TRUNC_VOX
NARROW = VS * NARROW_VOX
NSTEP = TRUNC_VOX + NARROW_VOX
NVOX = GS ** 3

NRAY = 65536
SP = 56
SENT = 0x3FFFFFFF
_BR = 4096


def _precompute_body(org_ref, dir_ref, dep_ref, flat_ref):
    ox, oy, oz = org_ref[0:1, :], org_ref[1:2, :], org_ref[2:3, :]
    dx, dy, dz = dir_ref[0:1, :], dir_ref[1:2, :], dir_ref[2:3, :]
    d = dep_ref[0:1, :]
    inv = 1.0 / (jnp.sqrt(dx * dx + dy * dy + dz * dz) + 1e-8)
    dx, dy, dz = dx * inv, dy * inv, dz * inv
    s = jax.lax.broadcasted_iota(jnp.int32, (SP, 1), 0)
    z = (d - TRUNC) + s.astype(jnp.float32) * VS
    px = ox + z * dx
    py = oy + z * dy
    pz = oz + z * dz
    inb = ((px > -1.0) & (px < 1.0) & (py > -1.0) & (py < 1.0)
           & (pz > -1.0) & (pz < 1.0) & (s < NSTEP))
    vx = jnp.clip(jnp.floor((px + 1.0) / VS).astype(jnp.int32), 0, GS - 1)
    vy = jnp.clip(jnp.floor((py + 1.0) / VS).astype(jnp.int32), 0, GS - 1)
    vz = jnp.clip(jnp.floor((pz + 1.0) / VS).astype(jnp.int32), 0, GS - 1)
    flat = (vx * GS + vy) * GS + vz
    flat_ref[...] = jnp.where(inb, flat, SENT)


def _precompute(origin, direction, depth):
    grid = (NRAY // _BR,)
    o3 = pl.BlockSpec((3, _BR), lambda i: (0, i))
    o1 = pl.BlockSpec((1, _BR), lambda i: (0, i))
    flat = pl.pallas_call(
        _precompute_body,
        grid=grid,
        in_specs=[o3, o3, o1],
        out_specs=pl.BlockSpec((SP, _BR), lambda i: (0, i)),
        out_shape=jax.ShapeDtypeStruct((SP, NRAY), jnp.int32),
    )(origin.T, direction.T, depth.T)
    return flat.reshape(-1)


NT = SP * NRAY
NRANGE = 32
VR = NVOX // NRANGE
BLK = 8192
NBLK = NT // BLK
PER_TILE = NBLK // 16
SLICE = VR // 16
FCH = 4096
ZB = 8192


def _step_consts(s):
    sf = jnp.full((16,), s, jnp.int32).astype(jnp.float32)
    sdf = TRUNC - sf * VS
    tsdf = jnp.clip(sdf / TRUNC, -1.0, 1.0)
    w = jnp.where(sdf >= 0.0, 1.0, jnp.clip(1.0 + sdf / NARROW, 0.0, 1.0))
    return w, tsdf * w


def _sc_body(fl_hbm, tg_hbm, wg_hbm, out_hbm,
             fl_buf, idx_st, w_st, t_st, gw, gt, bw, bt, zbuf,
             acc_w, acc_ts):
    from jax.experimental.pallas import tpu_sc as plsc
    c = lax.axis_index("c")
    sid = lax.axis_index("s")

    def zb_fill(i, _):
        zbuf[pl.ds(i * 16, 16)] = jnp.zeros((16,), jnp.float32)
        return 0
    lax.fori_loop(0, ZB // 16, zb_fill, 0)

    def pass_body(p, _):
        vbase = (c * 16 + p) * VR

        def zero1(k, _):
            off = sid * SLICE + k * ZB
            pltpu.sync_copy(zbuf, acc_w.at[pl.ds(off, ZB)])
            pltpu.sync_copy(zbuf, acc_ts.at[pl.ds(off, ZB)])
            return 0
        lax.fori_loop(0, SLICE // ZB, zero1, 0)
        plsc.subcore_barrier()

        def block_body(b, _):
            blk = sid * PER_TILE + b
            pltpu.sync_copy(fl_hbm.at[pl.ds(blk * BLK, BLK)], fl_buf)
            wc, tc = _step_consts(blk >> 3)

            def vloop(i, _):
                sl = pl.ds(i * 16, 16)
                li = fl_buf[sl] - vbase
                m = (li >= 0) & (li < VR)
                idx_st[sl] = jnp.where(m, li, VR)
                w_st[sl] = jnp.where(m, wc, 0.0)
                t_st[sl] = jnp.where(m, tc, 0.0)
                return 0
            lax.fori_loop(0, BLK // 16, vloop, 0)

            pltpu.sync_copy(w_st, acc_w.at[idx_st], add=True)
            pltpu.sync_copy(t_st, acc_ts.at[idx_st], add=True)
            return 0
        lax.fori_loop(0, PER_TILE, block_body, 0)
        plsc.subcore_barrier()

        def fin(k, _):
            loc = sid * SLICE + k * FCH
            g = vbase + loc
            pltpu.sync_copy(acc_w.at[pl.ds(loc, FCH)], gw)
            pltpu.sync_copy(acc_ts.at[pl.ds(loc, FCH)], gt)
            pltpu.sync_copy(wg_hbm.at[pl.ds(g, FCH)], bw)
            pltpu.sync_copy(tg_hbm.at[pl.ds(g, FCH)], bt)

            def vf(i, _):
                sl = pl.ds(i * 16, 16)
                wa = gw[sl]
                ta = gt[sl]
                wg0 = bw[sl]
                tg0 = bt[sl]
                nw = wg0 + wa
                nt = jnp.where(nw > 0.0,
                               (tg0 * wg0 + ta) / jnp.maximum(nw, 1e-8),
                               tg0)
                gw[sl] = nw
                gt[sl] = nt
                return 0
            lax.fori_loop(0, FCH // 16, vf, 0)
            pltpu.sync_copy(gt, out_hbm.at[pl.ds(g, FCH)])
            pltpu.sync_copy(gw, out_hbm.at[pl.ds(NVOX + g, FCH)])
            return 0
        lax.fori_loop(0, SLICE // FCH, fin, 0)
        plsc.subcore_barrier()
        return 0

    lax.fori_loop(0, NRANGE // 2, pass_body, 0)


def _sc_scatter(flat_f, tsdf_grid, weight_grid):
    from jax.experimental.pallas import tpu_sc as plsc
    mesh = plsc.VectorSubcoreMesh(core_axis_name="c", subcore_axis_name="s")
    f = functools.partial(
        pl.kernel, mesh=mesh,
        out_type=jax.ShapeDtypeStruct((2 * NVOX,), jnp.float32),
        scratch_types=[
            pltpu.VMEM((BLK,), jnp.int32),
            pltpu.VMEM((BLK,), jnp.int32),
            pltpu.VMEM((BLK,), jnp.float32),
            pltpu.VMEM((BLK,), jnp.float32),
            pltpu.VMEM((FCH,), jnp.float32),
            pltpu.VMEM((FCH,), jnp.float32),
            pltpu.VMEM((FCH,), jnp.float32),
            pltpu.VMEM((FCH,), jnp.float32),
            pltpu.VMEM((ZB,), jnp.float32),
            pltpu.VMEM_SHARED((VR + 16,), jnp.float32),
            pltpu.VMEM_SHARED((VR + 16,), jnp.float32),
        ],
    )(_sc_body)
    return f(flat_f, tsdf_grid, weight_grid)


def kernel(origin, direction, depth, tsdf_grid, weight_grid):
    flat_f = _precompute(origin, direction, depth)
    out = _sc_scatter(flat_f, tsdf_grid, weight_grid)
    return out.reshape(2, NVOX)


# speedup vs baseline: 15.9779x; 15.9779x over previous
import functools

import jax
import jax.numpy as jnp
from jax import lax
from jax.experimental import pallas as pl
from jax.experimental.pallas import tpu as pltpu

GS = 256
TRUNC_VOX = 39
NARROW_VOX = 13
VS = 2.0 / GS
TRUNC = VS * TRUNC_VOX
NARROW = VS * NARROW_VOX
NSTEP = TRUNC_VOX + NARROW_VOX
NVOX = GS ** 3

NRAY = 65536
SP = 56
SENT = 0x3FFFFFFF
_BR = 4096


def _precompute_body(org_ref, dir_ref, dep_ref, flat_ref):
    ox, oy, oz = org_ref[0:1, :], org_ref[1:2, :], org_ref[2:3, :]
    dx, dy, dz = dir_ref[0:1, :], dir_ref[1:2, :], dir_ref[2:3, :]
    d = dep_ref[0:1, :]
    inv = 1.0 / (jnp.sqrt(dx * dx + dy * dy + dz * dz) + 1e-8)
    dx, dy, dz = dx * inv, dy * inv, dz * inv
    s = jax.lax.broadcasted_iota(jnp.int32, (SP, 1), 0)
    z = (d - TRUNC) + s.astype(jnp.float32) * VS
    px = ox + z * dx
    py = oy + z * dy
    pz = oz + z * dz
    inb = ((px > -1.0) & (px < 1.0) & (py > -1.0) & (py < 1.0)
           & (pz > -1.0) & (pz < 1.0) & (s < NSTEP))
    vx = jnp.clip(jnp.floor((px + 1.0) / VS).astype(jnp.int32), 0, GS - 1)
    vy = jnp.clip(jnp.floor((py + 1.0) / VS).astype(jnp.int32), 0, GS - 1)
    vz = jnp.clip(jnp.floor((pz + 1.0) / VS).astype(jnp.int32), 0, GS - 1)
    flat = (vx * GS + vy) * GS + vz
    flat_ref[...] = jnp.where(inb, flat, SENT)


def _precompute(origin, direction, depth):
    grid = (NRAY // _BR,)
    o3 = pl.BlockSpec((3, _BR), lambda i: (0, i))
    o1 = pl.BlockSpec((1, _BR), lambda i: (0, i))
    flat = pl.pallas_call(
        _precompute_body,
        grid=grid,
        in_specs=[o3, o3, o1],
        out_specs=pl.BlockSpec((SP, _BR), lambda i: (0, i)),
        out_shape=jax.ShapeDtypeStruct((SP, NRAY), jnp.int32),
    )(origin.T, direction.T, depth.T)
    return flat.reshape(-1)


NT = SP * NRAY
NRANGE = 32
VR = NVOX // NRANGE
BLK = 8192
NBLK = NT // BLK
PER_TILE = NBLK // 16
SLICE = VR // 16
FCH = 4096
ZB = 8192


def _step_consts(s):
    sf = jnp.full((16,), s, jnp.int32).astype(jnp.float32)
    sdf = TRUNC - sf * VS
    tsdf = jnp.clip(sdf / TRUNC, -1.0, 1.0)
    w = jnp.where(sdf >= 0.0, 1.0, jnp.clip(1.0 + sdf / NARROW, 0.0, 1.0))
    return w, tsdf * w


def _sc_body(fl_hbm, tg_hbm, wg_hbm, out_hbm,
             fl_buf, idx_st, w_st, t_st, gw, gt, bw, bt, zbuf,
             acc_w, acc_ts):
    from jax.experimental.pallas import tpu_sc as plsc
    c = lax.axis_index("c")
    sid = lax.axis_index("s")

    def zb_fill(i, _):
        zbuf[pl.ds(i * 16, 16)] = jnp.zeros((16,), jnp.float32)
        return 0
    lax.fori_loop(0, ZB // 16, zb_fill, 0)

    def pass_body(p, _):
        vbase = (c * 16 + p) * VR

        def zero1(k, _):
            off = sid * SLICE + k * ZB
            pltpu.sync_copy(zbuf, acc_w.at[pl.ds(off, ZB)])
            pltpu.sync_copy(zbuf, acc_ts.at[pl.ds(off, ZB)])
            return 0
        lax.fori_loop(0, SLICE // ZB, zero1, 0)
        plsc.subcore_barrier()

        def block_body(b, _):
            blk = sid * PER_TILE + b
            pltpu.sync_copy(fl_hbm.at[pl.ds(blk * BLK, BLK)], fl_buf)
            wc, tc = _step_consts(blk >> 3)

            lane = lax.iota(jnp.int32, 16)

            def vloop(i, _):
                sl = pl.ds(i * 16, 16)
                li = fl_buf[sl] - vbase
                m = (li >= 0) & (li < VR)
                idx_st[sl] = jnp.where(m, li, VR + i * 16 + lane)
                w_st[sl] = jnp.where(m, wc, 0.0)
                t_st[sl] = jnp.where(m, tc, 0.0)
                return 0
            lax.fori_loop(0, BLK // 16, vloop, 0)

            pltpu.sync_copy(w_st, acc_w.at[idx_st], add=True)
            pltpu.sync_copy(t_st, acc_ts.at[idx_st], add=True)
            return 0
        lax.fori_loop(0, PER_TILE, block_body, 0)
        plsc.subcore_barrier()

        def fin(k, _):
            loc = sid * SLICE + k * FCH
            g = vbase + loc
            pltpu.sync_copy(acc_w.at[pl.ds(loc, FCH)], gw)
            pltpu.sync_copy(acc_ts.at[pl.ds(loc, FCH)], gt)
            pltpu.sync_copy(wg_hbm.at[pl.ds(g, FCH)], bw)
            pltpu.sync_copy(tg_hbm.at[pl.ds(g, FCH)], bt)

            def vf(i, _):
                sl = pl.ds(i * 16, 16)
                wa = gw[sl]
                ta = gt[sl]
                wg0 = bw[sl]
                tg0 = bt[sl]
                nw = wg0 + wa
                nt = jnp.where(nw > 0.0,
                               (tg0 * wg0 + ta) / jnp.maximum(nw, 1e-8),
                               tg0)
                gw[sl] = nw
                gt[sl] = nt
                return 0
            lax.fori_loop(0, FCH // 16, vf, 0)
            pltpu.sync_copy(gt, out_hbm.at[pl.ds(g, FCH)])
            pltpu.sync_copy(gw, out_hbm.at[pl.ds(NVOX + g, FCH)])
            return 0
        lax.fori_loop(0, SLICE // FCH, fin, 0)
        plsc.subcore_barrier()
        return 0

    lax.fori_loop(0, NRANGE // 2, pass_body, 0)


def _sc_scatter(flat_f, tsdf_grid, weight_grid):
    from jax.experimental.pallas import tpu_sc as plsc
    mesh = plsc.VectorSubcoreMesh(core_axis_name="c", subcore_axis_name="s")
    f = functools.partial(
        pl.kernel, mesh=mesh,
        out_type=jax.ShapeDtypeStruct((2 * NVOX,), jnp.float32),
        scratch_types=[
            pltpu.VMEM((BLK,), jnp.int32),
            pltpu.VMEM((BLK,), jnp.int32),
            pltpu.VMEM((BLK,), jnp.float32),
            pltpu.VMEM((BLK,), jnp.float32),
            pltpu.VMEM((FCH,), jnp.float32),
            pltpu.VMEM((FCH,), jnp.float32),
            pltpu.VMEM((FCH,), jnp.float32),
            pltpu.VMEM((FCH,), jnp.float32),
            pltpu.VMEM((ZB,), jnp.float32),
            pltpu.VMEM_SHARED((VR + BLK,), jnp.float32),
            pltpu.VMEM_SHARED((VR + BLK,), jnp.float32),
        ],
    )(_sc_body)
    return f(flat_f, tsdf_grid, weight_grid)


def kernel(origin, direction, depth, tsdf_grid, weight_grid):
    flat_f = _precompute(origin, direction, depth)
    out = _sc_scatter(flat_f, tsdf_grid, weight_grid)
    return out.reshape(2, NVOX)
